# baseline (speedup 1.0000x reference)
import jax
import jax.numpy as jnp
from jax import lax
from jax.experimental import pallas as pl
from jax.experimental.pallas import tpu as pltpu
from jax.experimental.pallas import tpu_sc as plsc

_B = 16384
_F = 26
_E = 16
_V = 1000000
_VP = 1000064

_NC = 2
_NS = 16
_NW = _NC * _NS
_BW = _B // _NW

_TC = 1792
_NCHUNK = _V // _TC
_TAIL = _V - _NCHUNK * _TC
_XW = _NCHUNK - (_NCHUNK // _NW) * _NW


def _transpose_body(wt_hbm, tail_hbm, wrm_hbm,
                    tbufA, tbufB, tobufA, tobufB, semA, semB, osemA, osemB):
    c = lax.axis_index("c")
    s = lax.axis_index("s")
    w = c * _NS + s
    lanes = lax.iota(jnp.int32, 16)

    nchunks = jnp.where(w < _XW, _NCHUNK // _NW + 1, _NCHUNK // _NW)

    def col0(j):
        return (j * _NW + w) * _TC

    def transpose_chunk(tbuf, tobuf, n):
        def q_body(qg, carry):
            q0 = qg * 16
            qv = q0 + lanes
            row_vec = lax.shift_right_logical(qv, 3)
            colbase = (qv & 7) * 16
            for e in range(_E):
                vec = tbuf[e, pl.ds(q0, 16)]
                plsc.store_scatter(tobuf, [row_vec, colbase + e], vec)
            return carry
        lax.fori_loop(0, n // 16, q_body, 0)

    def out_copy(tobuf, osem, n, cbase):
        r0 = pl.multiple_of(cbase * _E // 128, 8)
        pltpu.async_copy(tobuf.at[pl.ds(0, n * _E // 128), :],
                         wrm_hbm.at[pl.ds(r0, n * _E // 128), :], osem)

    def out_wait(tobuf, osem, n):
        pltpu.make_async_copy(tobuf.at[pl.ds(0, n * _E // 128), :],
                              wrm_hbm.at[pl.ds(0, n * _E // 128), :], osem).wait()

    pltpu.async_copy(wt_hbm.at[:, pl.ds(col0(0), _TC)], tbufA, semA)

    def pair_body(j, carry):
        f0 = 2 * j

        @pl.when(f0 < nchunks)
        def _():
            @pl.when(f0 + 1 < nchunks)
            def _():
                pltpu.async_copy(wt_hbm.at[:, pl.ds(col0(f0 + 1), _TC)],
                                 tbufB, semB)
            pltpu.make_async_copy(wt_hbm.at[:, pl.ds(0, _TC)], tbufA, semA).wait()

            @pl.when(j > 0)
            def _():
                out_wait(tobufA, osemA, _TC)
            transpose_chunk(tbufA, tobufA, _TC)
            out_copy(tobufA, osemA, _TC, col0(f0))

            @pl.when(f0 + 2 < nchunks)
            def _():
                pltpu.async_copy(wt_hbm.at[:, pl.ds(col0(f0 + 2), _TC)],
                                 tbufA, semA)

        @pl.when(f0 + 1 < nchunks)
        def _():
            pltpu.make_async_copy(wt_hbm.at[:, pl.ds(0, _TC)], tbufB, semB).wait()

            @pl.when(j > 0)
            def _():
                out_wait(tobufB, osemB, _TC)
            transpose_chunk(tbufB, tobufB, _TC)
            out_copy(tobufB, osemB, _TC, col0(f0 + 1))
        return carry

    lax.fori_loop(0, (_NCHUNK // _NW + 2) // 2, pair_body, 0)
    out_wait(tobufA, osemA, _TC)
    out_wait(tobufB, osemB, _TC)

    @pl.when(w == _XW)
    def _():
        pltpu.sync_copy(tail_hbm, tbufA)
        transpose_chunk(tbufA, tobufA, _TAIL)
        out_copy(tobufA, osemA, _TAIL, _NCHUNK * _TC)
        out_wait(tobufA, osemA, _TAIL)


def _lookup_body(idT_hbm, valT_hbm, wrm_hbm, out_hbm,
                 idbuf, valbuf, rowsA, rowsB, obufA, obufB,
                 gsem0, gsem1, osemA, osemB):
    c = lax.axis_index("c")
    s = lax.axis_index("s")
    wid = c * _NS + s
    b0 = wid * _BW
    lanes = lax.iota(jnp.int32, 16)
    rows2 = (rowsA, rowsB)
    gsems = (gsem0, gsem1)
    obufs = (obufA, obufB)
    osems = (osemA, osemB)

    pltpu.sync_copy(idT_hbm.at[:, pl.ds(b0, _BW)], idbuf)
    pltpu.sync_copy(valT_hbm.at[:, pl.ds(b0, _BW)], valbuf)

    def gather_r(r, k):
        pltpu.async_copy(wrm_hbm.at[idbuf.at[2 * r]], rows2[k].at[0], gsems[k])
        pltpu.async_copy(wrm_hbm.at[idbuf.at[2 * r + 1]], rows2[k].at[1], gsems[k])

    def gwait(k):
        for j in range(2):
            pltpu.make_async_copy(wrm_hbm.at[idbuf.at[0]],
                                  rows2[k].at[j], gsems[k]).wait()

    def owait(m):
        pltpu.make_async_copy(obufs[m],
                              out_hbm.at[pl.ds(0, 2), :, pl.ds(b0, _BW)],
                              osems[m]).wait()

    def compute_r(r, k, m):
        obuf = obufs[m]
        rows = rows2[k]

        def g_body(g, carry):
            j = g // 32
            gg = g - j * 32
            f = 2 * r + j
            jv = jnp.full((16,), j, jnp.int32)
            bidx = gg * 16 + lanes
            v16 = valbuf[f, pl.ds(gg * 16, 16)]
            for e in range(_E):
                col = jnp.full((16,), e, jnp.int32)
                obuf[j, e, pl.ds(gg * 16, 16)] = (
                    plsc.load_gather(rows, [jv, bidx, col]) * v16)
            return carry
        lax.fori_loop(0, 2 * (_BW // 16), g_body, 0)
        pltpu.async_copy(obuf, out_hbm.at[pl.ds(2 * r, 2), :, pl.ds(b0, _BW)],
                         osems[m])

    _NR = _F // 2
    gather_r(0, 0)

    def r_pair(p, carry):
        rA = 2 * p
        gather_r(rA + 1, 1)
        gwait(0)

        @pl.when(p > 0)
        def _():
            owait(0)
        compute_r(rA, 0, 0)

        @pl.when(rA + 2 < _NR)
        def _():
            gather_r(rA + 2, 0)

        gwait(1)

        @pl.when(p > 0)
        def _():
            owait(1)
        compute_r(rA + 1, 1, 1)
        return carry

    lax.fori_loop(0, (_NR - 1) // 2, r_pair, 0)
    gwait(0)
    owait(0)
    compute_r(_NR - 1, 0, 0)
    owait(0)
    owait(1)


def kernel(id, value, W):
    idT = id.T
    valT = value.T
    wT = W.T
    tail = jnp.pad(lax.slice(wT, (0, _NCHUNK * _TC), (_E, _V)),
                   ((0, 0), (0, _TC - _TAIL)))
    mesh = plsc.VectorSubcoreMesh(core_axis_name="c", subcore_axis_name="s")

    wrm128 = pl.kernel(
        _transpose_body,
        mesh=mesh,
        compiler_params=pltpu.CompilerParams(
            use_tc_tiling_on_sc=True, needs_layout_passes=False),
        out_type=jax.ShapeDtypeStruct((_VP * _E // 128, 128), jnp.float32),
        scratch_types=[
            pltpu.VMEM((_E, _TC), jnp.float32),
            pltpu.VMEM((_E, _TC), jnp.float32),
            pltpu.VMEM((_TC * _E // 128, 128), jnp.float32),
            pltpu.VMEM((_TC * _E // 128, 128), jnp.float32),
            pltpu.SemaphoreType.DMA,
            pltpu.SemaphoreType.DMA,
            pltpu.SemaphoreType.DMA,
            pltpu.SemaphoreType.DMA,
        ],
    )(wT, tail)
    wrm = wrm128.reshape(_VP, _E)

    outT = pl.kernel(
        _lookup_body,
        mesh=mesh,
        compiler_params=pltpu.CompilerParams(
            use_tc_tiling_on_sc=False, needs_layout_passes=False),
        out_type=jax.ShapeDtypeStruct((_F, _E, _B), jnp.float32),
        scratch_types=[
            pltpu.VMEM((_F, _BW), jnp.int32),
            pltpu.VMEM((_F, _BW), jnp.float32),
            pltpu.VMEM((2, _BW, _E), jnp.float32),
            pltpu.VMEM((2, _BW, _E), jnp.float32),
            pltpu.VMEM((2, _E, _BW), jnp.float32),
            pltpu.VMEM((2, _E, _BW), jnp.float32),
            pltpu.SemaphoreType.DMA,
            pltpu.SemaphoreType.DMA,
            pltpu.SemaphoreType.DMA,
            pltpu.SemaphoreType.DMA,
        ],
    )(idT, valT, wrm)
    return outT.transpose(2, 0, 1)


# speedup vs baseline: 1.1150x; 1.1150x over previous
import jax
import jax.numpy as jnp
from jax import lax
from jax.experimental import pallas as pl
from jax.experimental.pallas import tpu as pltpu
from jax.experimental.pallas import tpu_sc as plsc

_B = 16384
_F = 26
_E = 16
_V = 1000000
_VP = 1000064

_NC = 2
_NS = 16
_NW = _NC * _NS
_BW = _B // _NW

_TC = 1792
_NCHUNK = _V // _TC
_TAIL = _V - _NCHUNK * _TC
_XW = _NCHUNK - (_NCHUNK // _NW) * _NW


def _transpose_body(wt_hbm, tail_hbm, wrm_hbm,
                    tbufA, tbufB, tobufA, tobufB, semA, semB, osemA, osemB):
    c = lax.axis_index("c")
    s = lax.axis_index("s")
    w = c * _NS + s
    lanes = lax.iota(jnp.int32, 16)

    nchunks = jnp.where(w < _XW, _NCHUNK // _NW + 1, _NCHUNK // _NW)

    def col0(j):
        return (j * _NW + w) * _TC

    def transpose_chunk(tbuf, tobuf, n):
        def q_body(qg, carry):
            q0 = qg * 16
            qv = q0 + lanes
            row_vec = lax.shift_right_logical(qv, 3)
            colbase = (qv & 7) * 16
            for e in range(_E):
                vec = tbuf[e, pl.ds(q0, 16)]
                plsc.store_scatter(tobuf, [row_vec, colbase + e], vec)
            return carry
        lax.fori_loop(0, n // 16, q_body, 0)

    def out_copy(tobuf, osem, n, cbase):
        r0 = pl.multiple_of(cbase * _E // 128, 8)
        pltpu.async_copy(tobuf.at[pl.ds(0, n * _E // 128), :],
                         wrm_hbm.at[pl.ds(r0, n * _E // 128), :], osem)

    def out_wait(tobuf, osem, n):
        pltpu.make_async_copy(tobuf.at[pl.ds(0, n * _E // 128), :],
                              wrm_hbm.at[pl.ds(0, n * _E // 128), :], osem).wait()

    pltpu.async_copy(wt_hbm.at[:, pl.ds(col0(0), _TC)], tbufA, semA)

    def pair_body(j, carry):
        f0 = 2 * j

        @pl.when(f0 < nchunks)
        def _():
            @pl.when(f0 + 1 < nchunks)
            def _():
                pltpu.async_copy(wt_hbm.at[:, pl.ds(col0(f0 + 1), _TC)],
                                 tbufB, semB)
            pltpu.make_async_copy(wt_hbm.at[:, pl.ds(0, _TC)], tbufA, semA).wait()

            @pl.when(j > 0)
            def _():
                out_wait(tobufA, osemA, _TC)
            transpose_chunk(tbufA, tobufA, _TC)
            out_copy(tobufA, osemA, _TC, col0(f0))

            @pl.when(f0 + 2 < nchunks)
            def _():
                pltpu.async_copy(wt_hbm.at[:, pl.ds(col0(f0 + 2), _TC)],
                                 tbufA, semA)

        @pl.when(f0 + 1 < nchunks)
        def _():
            pltpu.make_async_copy(wt_hbm.at[:, pl.ds(0, _TC)], tbufB, semB).wait()

            @pl.when(j > 0)
            def _():
                out_wait(tobufB, osemB, _TC)
            transpose_chunk(tbufB, tobufB, _TC)
            out_copy(tobufB, osemB, _TC, col0(f0 + 1))
        return carry

    lax.fori_loop(0, (_NCHUNK // _NW + 2) // 2, pair_body, 0)
    out_wait(tobufA, osemA, _TC)
    out_wait(tobufB, osemB, _TC)

    @pl.when(w == _XW)
    def _():
        pltpu.sync_copy(tail_hbm, tbufA)
        transpose_chunk(tbufA, tobufA, _TAIL)
        out_copy(tobufA, osemA, _TAIL, _NCHUNK * _TC)
        out_wait(tobufA, osemA, _TAIL)


def _lookup_body(idT_hbm, valT_hbm, wrm_hbm, out_hbm,
                 idbuf, valbuf, rowsA, rowsB, obufA, obufB,
                 gsem0, gsem1, osemA, osemB):
    c = lax.axis_index("c")
    s = lax.axis_index("s")
    wid = c * _NS + s
    b0 = wid * _BW
    lanes = lax.iota(jnp.int32, 16)
    rows2 = (rowsA, rowsB)
    gsems = (gsem0, gsem1)
    obufs = (obufA, obufB)
    osems = (osemA, osemB)

    pltpu.sync_copy(idT_hbm.at[:, pl.ds(b0, _BW)], idbuf)
    pltpu.sync_copy(valT_hbm.at[:, pl.ds(b0, _BW)], valbuf)

    def gather_r(r, k):
        pltpu.async_copy(wrm_hbm.at[idbuf.at[2 * r]], rows2[k].at[0], gsems[k])
        pltpu.async_copy(wrm_hbm.at[idbuf.at[2 * r + 1]], rows2[k].at[1], gsems[k])

    def gwait(k):
        for j in range(2):
            pltpu.make_async_copy(wrm_hbm.at[idbuf.at[0]],
                                  rows2[k].at[j], gsems[k]).wait()

    def owait(m):
        pltpu.make_async_copy(
            obufs[m],
            out_hbm.at[pl.ds(0, 2), :, pl.ds(0, _BW // 128), :, :],
            osems[m]).wait()

    def compute_r(r, k, m):
        obuf = obufs[m]
        rows = rows2[k]

        def g_body(g, carry):
            j = g // 32
            gg = g - j * 32
            f = 2 * r + j
            tb = gg // 8
            j0 = (gg - tb * 8) * 16
            jv = jnp.full((16,), j, jnp.int32)
            bidx = gg * 16 + lanes
            v16 = valbuf[f, pl.ds(gg * 16, 16)]
            for e in range(_E):
                col = jnp.full((16,), e, jnp.int32)
                obuf[j, e // 8, tb, e % 8, pl.ds(j0, 16)] = (
                    plsc.load_gather(rows, [jv, bidx, col]) * v16)
            return carry
        lax.fori_loop(0, 2 * (_BW // 16), g_body, 0)
        pltpu.async_copy(
            obuf,
            out_hbm.at[pl.ds(2 * r, 2), :, pl.ds(b0 // 128, _BW // 128), :, :],
            osems[m])

    _NR = _F // 2
    gather_r(0, 0)

    def r_pair(p, carry):
        rA = 2 * p
        gather_r(rA + 1, 1)
        gwait(0)

        @pl.when(p > 0)
        def _():
            owait(0)
        compute_r(rA, 0, 0)

        @pl.when(rA + 2 < _NR)
        def _():
            gather_r(rA + 2, 0)

        gwait(1)

        @pl.when(p > 0)
        def _():
            owait(1)
        compute_r(rA + 1, 1, 1)
        return carry

    lax.fori_loop(0, (_NR - 1) // 2, r_pair, 0)
    gwait(0)
    owait(0)
    compute_r(_NR - 1, 0, 0)
    owait(0)
    owait(1)


def kernel(id, value, W):
    idT = id.T
    valT = value.T
    wT = W.T
    tail = jnp.pad(lax.slice(wT, (0, _NCHUNK * _TC), (_E, _V)),
                   ((0, 0), (0, _TC - _TAIL)))
    mesh = plsc.VectorSubcoreMesh(core_axis_name="c", subcore_axis_name="s")

    wrm128 = pl.kernel(
        _transpose_body,
        mesh=mesh,
        compiler_params=pltpu.CompilerParams(
            use_tc_tiling_on_sc=True, needs_layout_passes=False),
        out_type=jax.ShapeDtypeStruct((_VP * _E // 128, 128), jnp.float32),
        scratch_types=[
            pltpu.VMEM((_E, _TC), jnp.float32),
            pltpu.VMEM((_E, _TC), jnp.float32),
            pltpu.VMEM((_TC * _E // 128, 128), jnp.float32),
            pltpu.VMEM((_TC * _E // 128, 128), jnp.float32),
            pltpu.SemaphoreType.DMA,
            pltpu.SemaphoreType.DMA,
            pltpu.SemaphoreType.DMA,
            pltpu.SemaphoreType.DMA,
        ],
    )(wT, tail)
    wrm = wrm128.reshape(_VP, _E)

    outL = pl.kernel(
        _lookup_body,
        mesh=mesh,
        compiler_params=pltpu.CompilerParams(
            use_tc_tiling_on_sc=False, needs_layout_passes=False),
        out_type=jax.ShapeDtypeStruct((_F, 2, _B // 128, 8, 128), jnp.float32),
        scratch_types=[
            pltpu.VMEM((_F, _BW), jnp.int32),
            pltpu.VMEM((_F, _BW), jnp.float32),
            pltpu.VMEM((2, _BW, _E), jnp.float32),
            pltpu.VMEM((2, _BW, _E), jnp.float32),
            pltpu.VMEM((2, 2, _BW // 128, 8, 128), jnp.float32),
            pltpu.VMEM((2, 2, _BW // 128, 8, 128), jnp.float32),
            pltpu.SemaphoreType.DMA,
            pltpu.SemaphoreType.DMA,
            pltpu.SemaphoreType.DMA,
            pltpu.SemaphoreType.DMA,
        ],
    )(idT, valT, wrm)
    return outL.transpose(2, 4, 0, 1, 3).reshape(_B, _F, _E)


# speedup vs baseline: 1.1191x; 1.0037x over previous
import jax
import jax.numpy as jnp
from jax import lax
from jax.experimental import pallas as pl
from jax.experimental.pallas import tpu as pltpu
from jax.experimental.pallas import tpu_sc as plsc

_B = 16384
_F = 26
_E = 16
_V = 1000000
_VP = 1000064

_NC = 2
_NS = 16
_NW = _NC * _NS
_BW = _B // _NW

_TC = 1792
_NCHUNK = _V // _TC
_TAIL = _V - _NCHUNK * _TC
_XW = _NCHUNK - (_NCHUNK // _NW) * _NW


def _transpose_body(wt_hbm, tail_hbm, wrm_hbm,
                    tbufA, tbufB, tobufA, tobufB, semA, semB, osemA, osemB):
    c = lax.axis_index("c")
    s = lax.axis_index("s")
    w = c * _NS + s
    lanes = lax.iota(jnp.int32, 16)

    nchunks = jnp.where(w < _XW, _NCHUNK // _NW + 1, _NCHUNK // _NW)

    def col0(j):
        return (j * _NW + w) * _TC

    def transpose_chunk(tbuf, tobuf, n):
        def q_body(qg, carry):
            q0 = qg * 16
            qv = q0 + lanes
            row_vec = lax.shift_right_logical(qv, 3)
            colbase = (qv & 7) * 16
            for e in range(_E):
                vec = tbuf[e, pl.ds(q0, 16)]
                plsc.store_scatter(tobuf, [row_vec, colbase + e], vec)
            return carry
        lax.fori_loop(0, n // 16, q_body, 0)

    def out_copy(tobuf, osem, n, cbase):
        r0 = pl.multiple_of(cbase * _E // 128, 8)
        pltpu.async_copy(tobuf.at[pl.ds(0, n * _E // 128), :],
                         wrm_hbm.at[pl.ds(r0, n * _E // 128), :], osem)

    def out_wait(tobuf, osem, n):
        pltpu.make_async_copy(tobuf.at[pl.ds(0, n * _E // 128), :],
                              wrm_hbm.at[pl.ds(0, n * _E // 128), :], osem).wait()

    pltpu.async_copy(wt_hbm.at[:, pl.ds(col0(0), _TC)], tbufA, semA)

    def pair_body(j, carry):
        f0 = 2 * j

        @pl.when(f0 < nchunks)
        def _():
            @pl.when(f0 + 1 < nchunks)
            def _():
                pltpu.async_copy(wt_hbm.at[:, pl.ds(col0(f0 + 1), _TC)],
                                 tbufB, semB)
            pltpu.make_async_copy(wt_hbm.at[:, pl.ds(0, _TC)], tbufA, semA).wait()

            @pl.when(j > 0)
            def _():
                out_wait(tobufA, osemA, _TC)
            transpose_chunk(tbufA, tobufA, _TC)
            out_copy(tobufA, osemA, _TC, col0(f0))

            @pl.when(f0 + 2 < nchunks)
            def _():
                pltpu.async_copy(wt_hbm.at[:, pl.ds(col0(f0 + 2), _TC)],
                                 tbufA, semA)

        @pl.when(f0 + 1 < nchunks)
        def _():
            pltpu.make_async_copy(wt_hbm.at[:, pl.ds(0, _TC)], tbufB, semB).wait()

            @pl.when(j > 0)
            def _():
                out_wait(tobufB, osemB, _TC)
            transpose_chunk(tbufB, tobufB, _TC)
            out_copy(tobufB, osemB, _TC, col0(f0 + 1))
        return carry

    lax.fori_loop(0, (_NCHUNK // _NW + 2) // 2, pair_body, 0)
    out_wait(tobufA, osemA, _TC)
    out_wait(tobufB, osemB, _TC)

    @pl.when(w == _XW)
    def _():
        pltpu.sync_copy(tail_hbm, tbufA)
        transpose_chunk(tbufA, tobufA, _TAIL)
        out_copy(tobufA, osemA, _TAIL, _NCHUNK * _TC)
        out_wait(tobufA, osemA, _TAIL)


def _lookup_body(idT_hbm, valT_hbm, wrm_hbm, out_hbm,
                 idbuf, valbuf, rowsA, rowsB, obufA, obufB,
                 gsem0, gsem1, osemA, osemB):
    c = lax.axis_index("c")
    s = lax.axis_index("s")
    wid = c * _NS + s
    b0 = wid * _BW
    lanes = lax.iota(jnp.int32, 16)
    rows2 = (rowsA, rowsB)
    gsems = (gsem0, gsem1)
    obufs = (obufA, obufB)
    osems = (osemA, osemB)

    pltpu.sync_copy(idT_hbm.at[:, pl.ds(b0, _BW)], idbuf)
    pltpu.sync_copy(valT_hbm.at[:, pl.ds(b0, _BW)], valbuf)

    def gather_r(r, k):
        pltpu.async_copy(wrm_hbm.at[idbuf.at[2 * r]], rows2[k].at[0], gsems[k])
        pltpu.async_copy(wrm_hbm.at[idbuf.at[2 * r + 1]], rows2[k].at[1], gsems[k])

    def gwait(k):
        for j in range(2):
            pltpu.make_async_copy(wrm_hbm.at[idbuf.at[0]],
                                  rows2[k].at[j], gsems[k]).wait()

    def owait(m):
        pltpu.make_async_copy(
            obufs[m],
            out_hbm.at[pl.ds(0, 2), :, pl.ds(0, _BW // 128), :, :],
            osems[m]).wait()

    def compute_r(r, k, m):
        obuf = obufs[m]
        rows = rows2[k]

        def g_body(g, carry):
            j = g // 32
            gg = g - j * 32
            f = 2 * r + j
            tb = gg // 8
            j0 = (gg - tb * 8) * 16
            jv = jnp.full((16,), j, jnp.int32)
            bidx = gg * 16 + lanes
            v16 = valbuf[f, pl.ds(gg * 16, 16)]
            for e in range(_E):
                col = jnp.full((16,), e, jnp.int32)
                obuf[j, e // 8, tb, e % 8, pl.ds(j0, 16)] = (
                    plsc.load_gather(rows, [jv, bidx, col]) * v16)
            return carry
        lax.fori_loop(0, 2 * (_BW // 16), g_body, 0)
        pltpu.async_copy(
            obuf,
            out_hbm.at[pl.ds(2 * r, 2), :, pl.ds(b0 // 128, _BW // 128), :, :],
            osems[m])

    _NR = _F // 2
    gather_r(0, 0)

    def r_pair(p, carry):
        rA = 2 * p
        gather_r(rA + 1, 1)
        gwait(0)

        @pl.when(p > 0)
        def _():
            owait(0)
        compute_r(rA, 0, 0)

        @pl.when(rA + 2 < _NR)
        def _():
            gather_r(rA + 2, 0)

        gwait(1)

        @pl.when(p > 0)
        def _():
            owait(1)
        compute_r(rA + 1, 1, 1)
        return carry

    lax.fori_loop(0, (_NR - 1) // 2, r_pair, 0)
    gwait(0)
    owait(0)
    compute_r(_NR - 1, 0, 0)
    owait(0)
    owait(1)


def kernel(id, value, W):
    id = id.astype(jnp.int32)
    idT = id.T
    valT = value.T
    wT = W.T
    tail = jnp.pad(lax.slice(wT, (0, _NCHUNK * _TC), (_E, _V)),
                   ((0, 0), (0, _TC - _TAIL)))
    mesh = plsc.VectorSubcoreMesh(core_axis_name="c", subcore_axis_name="s")

    wrm128 = pl.kernel(
        _transpose_body,
        mesh=mesh,
        compiler_params=pltpu.CompilerParams(
            use_tc_tiling_on_sc=True, needs_layout_passes=False),
        out_type=jax.ShapeDtypeStruct((_VP * _E // 128, 128), jnp.float32),
        scratch_types=[
            pltpu.VMEM((_E, _TC), jnp.float32),
            pltpu.VMEM((_E, _TC), jnp.float32),
            pltpu.VMEM((_TC * _E // 128, 128), jnp.float32),
            pltpu.VMEM((_TC * _E // 128, 128), jnp.float32),
            pltpu.SemaphoreType.DMA,
            pltpu.SemaphoreType.DMA,
            pltpu.SemaphoreType.DMA,
            pltpu.SemaphoreType.DMA,
        ],
    )(wT, tail)
    wrm = wrm128.reshape(_VP, _E)

    outL = pl.kernel(
        _lookup_body,
        mesh=mesh,
        compiler_params=pltpu.CompilerParams(
            use_tc_tiling_on_sc=False, needs_layout_passes=False),
        out_type=jax.ShapeDtypeStruct((_F, 2, _B // 128, 8, 128), jnp.float32),
        scratch_types=[
            pltpu.VMEM((_F, _BW), jnp.int32),
            pltpu.VMEM((_F, _BW), jnp.float32),
            pltpu.VMEM((2, _BW, _E), jnp.float32),
            pltpu.VMEM((2, _BW, _E), jnp.float32),
            pltpu.VMEM((2, 2, _BW // 128, 8, 128), jnp.float32),
            pltpu.VMEM((2, 2, _BW // 128, 8, 128), jnp.float32),
            pltpu.SemaphoreType.DMA,
            pltpu.SemaphoreType.DMA,
            pltpu.SemaphoreType.DMA,
            pltpu.SemaphoreType.DMA,
        ],
    )(idT, valT, wrm)
    return outL.transpose(2, 4, 0, 1, 3).reshape(_B, _F, _E)
